# Initial kernel scaffold; baseline (speedup 1.0000x reference)
import jax
import jax.numpy as jnp
from jax.experimental import pallas as pl


def kernel(x, edge_index, conv0_W1, conv0_b1, conv0_W2, conv0_b2, conv1_W1, conv1_b1, conv1_W2, conv1_b2, lin_W, lin_b):
    raise NotImplementedError("write your pallas kernel here")


# speedup vs baseline: 7.1353x; 7.1353x over previous
import functools

import jax
import jax.numpy as jnp
from jax import lax
from jax.experimental import pallas as pl
from jax.experimental.pallas import tpu as pltpu
from jax.experimental.pallas import tpu_sc as plsc

N_NODES = 10000
NPAD = 10240
N_EDGES = 320000
D = 128

NC = 2
NS = 16
NW = NC * NS
EPW = N_EDGES // NW
K = 80
NCHUNK = EPW // K
RPT = NPAD // NS


def _agg_body(h_hbm, src_hbm, dst_hbm, zero_hbm, out_hbm,
              src_v, dst_v, rows_v, acc_sh, sem):
    c = lax.axis_index("c")
    s = lax.axis_index("s")
    wid = s * NC + c
    row0 = s * RPT

    @pl.when(c == 0)
    def _():
        pltpu.sync_copy(h_hbm.at[pl.ds(row0, RPT)], acc_sh.at[pl.ds(row0, RPT)])

    @pl.when(c != 0)
    def _():
        pltpu.sync_copy(zero_hbm.at[pl.ds(row0, RPT)], acc_sh.at[pl.ds(row0, RPT)])

    pltpu.sync_copy(src_hbm.at[wid], src_v)
    pltpu.sync_copy(dst_hbm.at[wid], dst_v)
    plsc.subcore_barrier()

    def body(i, carry):
        pltpu.async_copy(h_hbm.at[src_v.at[i]], rows_v, sem).wait()
        pltpu.sync_copy(rows_v, acc_sh.at[dst_v.at[i]], add=True)
        return carry

    lax.fori_loop(0, NCHUNK, body, 0)
    plsc.subcore_barrier()
    pltpu.sync_copy(acc_sh.at[pl.ds(row0, RPT)],
                    out_hbm.at[c].at[pl.ds(row0, RPT)])


_gin_agg = functools.partial(
    pl.kernel,
    out_type=jax.ShapeDtypeStruct((NC, NPAD, D), jnp.float32),
    mesh=plsc.VectorSubcoreMesh(core_axis_name="c", subcore_axis_name="s",
                                num_cores=NC, num_subcores=NS),
    scratch_types=[
        pltpu.VMEM((NCHUNK, K), jnp.int32),
        pltpu.VMEM((NCHUNK, K), jnp.int32),
        pltpu.VMEM((K, D), jnp.float32),
        pltpu.VMEM_SHARED((NPAD, D), jnp.float32),
        pltpu.SemaphoreType.DMA,
    ],
)(_agg_body)


R = 1000


def _elu(x):
    return jnp.where(x > 0, x, jnp.exp(jnp.minimum(x, 0.0)) - 1.0)


def _mlp_mid_body(p_ref, W1_ref, b1_ref, W2_ref, b2_ref, out_ref):
    z = p_ref[0] + p_ref[1]
    z = jnp.maximum(
        jnp.dot(z, W1_ref[...], preferred_element_type=jnp.float32)
        + b1_ref[...], 0.0)
    h = jnp.dot(z, W2_ref[...], preferred_element_type=jnp.float32) + b2_ref[...]
    out_ref[...] = _elu(h)


def _mlp_final_body(p_ref, W1_ref, b1_ref, W2_ref, b2_ref,
                    lw_ref, lb_ref, out_ref):
    z = p_ref[0] + p_ref[1]
    z = jnp.maximum(
        jnp.dot(z, W1_ref[...], preferred_element_type=jnp.float32)
        + b1_ref[...], 0.0)
    h = jnp.dot(z, W2_ref[...], preferred_element_type=jnp.float32) + b2_ref[...]
    h = _elu(h)
    o = jnp.dot(h, lw_ref[...], preferred_element_type=jnp.float32) + lb_ref[...]
    out_ref[...] = 1.0 / (1.0 + jnp.exp(-o))


_P_SPEC = pl.BlockSpec((NC, R, D), lambda i: (0, i, 0))
_W_SPEC = pl.BlockSpec((D, D), lambda i: (0, 0))
_B_SPEC = pl.BlockSpec((1, D), lambda i: (0, 0))

_mlp_mid = pl.pallas_call(
    _mlp_mid_body,
    grid=(N_NODES // R,),
    in_specs=[_P_SPEC, _W_SPEC, _B_SPEC, _W_SPEC, _B_SPEC],
    out_specs=pl.BlockSpec((R, D), lambda i: (i, 0)),
    out_shape=jax.ShapeDtypeStruct((NPAD, D), jnp.float32),
)

_mlp_final = pl.pallas_call(
    _mlp_final_body,
    grid=(N_NODES // R,),
    in_specs=[_P_SPEC, _W_SPEC, _B_SPEC, _W_SPEC, _B_SPEC,
              pl.BlockSpec((D, 1), lambda i: (0, 0)),
              pl.BlockSpec((1, 1), lambda i: (0, 0))],
    out_specs=pl.BlockSpec((R, 1), lambda i: (i, 0)),
    out_shape=jax.ShapeDtypeStruct((N_NODES, 1), jnp.float32),
)


def kernel(x, edge_index, conv0_W1, conv0_b1, conv0_W2, conv0_b2,
           conv1_W1, conv1_b1, conv1_W2, conv1_b2, lin_W, lin_b):
    src = edge_index[0].astype(jnp.int32).reshape(NW, NCHUNK, K)
    dst = edge_index[1].astype(jnp.int32).reshape(NW, NCHUNK, K)
    zeros = jnp.zeros((NPAD, D), jnp.float32)
    x_pad = jnp.concatenate(
        [x, jnp.zeros((NPAD - N_NODES, D), jnp.float32)], axis=0)

    p = _gin_agg(x_pad, src, dst, zeros)
    h1 = _mlp_mid(p, conv0_W1, conv0_b1.reshape(1, D),
                  conv0_W2, conv0_b2.reshape(1, D))
    p = _gin_agg(h1, src, dst, zeros)
    out = _mlp_final(p, conv1_W1, conv1_b1.reshape(1, D),
                     conv1_W2, conv1_b2.reshape(1, D),
                     lin_W, lin_b.reshape(1, 1))
    return out.reshape(N_NODES)
